# baseline (speedup 1.0000x reference)
import functools

import jax
import jax.numpy as jnp
from jax import lax
from jax.experimental import pallas as pl
from jax.experimental.pallas import tpu as pltpu
from jax.experimental.pallas import tpu_sc as plsc

N = 10000
D = 256
E = 160000
BN = 400
NB = N // BN
NPAD = 10112
RW = NPAD // 16
CH = 80
PERW = E // 16
NCH = PERW // CH

def _mesh():
    return plsc.VectorSubcoreMesh(
        core_axis_name="c", subcore_axis_name="s", num_cores=2, num_subcores=16
    )


ECH = 176
NCHK = PERW // ECH
TAILE = PERW - NCHK * ECH


@functools.lru_cache(maxsize=None)
def _make_segsum():
    out_type = [jax.ShapeDtypeStruct((2 * NPAD, 128), jnp.float32)] * 3
    scratch = (
        [pltpu.VMEM((ECH,), jnp.int32) for _ in range(2)]
        + [pltpu.VMEM((ECH,), jnp.int32) for _ in range(2)]
        + [pltpu.VMEM((ECH, 128), jnp.float32) for _ in range(2)]
        + [pltpu.VMEM((TAILE,), jnp.int32) for _ in range(2)]
        + [pltpu.SemaphoreType.DMA for _ in range(4)]
        + [pltpu.VMEM_SHARED((NPAD, 128), jnp.float32)]
    )

    @functools.partial(
        pl.kernel, out_type=out_type, mesh=_mesh(), scratch_types=scratch
    )
    def k(yt0, yt1, yt2, es0, es1, es2, ed0, ed1, ed2, zrow,
          so0, so1, so2, *rest):
        sidx = rest[0:2]
        didx = rest[2:4]
        rows = rest[4:6]
        sidxt, didxt = rest[6:8]
        gsem = rest[8:10]
        ssem = rest[10:12]
        acc = rest[12]
        c = lax.axis_index("c")
        s = lax.axis_index("s")
        base = s * PERW
        coff = jnp.zeros((16,), jnp.int32) + c * N

        for ytab, es, ed, s_out in ((yt0, es0, ed0, so0),
                                    (yt1, es1, ed1, so1),
                                    (yt2, es2, ed2, so2)):
            pltpu.sync_copy(zrow, acc.at[pl.ds(s * RW, RW)])
            plsc.subcore_barrier()

            def load_idx(bb, j, es=es, ed=ed):
                off = pl.multiple_of(base + j * ECH, 16)
                pltpu.sync_copy(es.at[pl.ds(off, ECH)], sidx[bb])
                pltpu.sync_copy(ed.at[pl.ds(off, ECH)], didx[bb])

                def adj(r, carry2):
                    sl = pl.ds(r * 16, 16)
                    sidx[bb][sl] = sidx[bb][sl] + coff
                    return carry2

                lax.fori_loop(0, ECH // 16, adj, 0)

            load_idx(0, 0)
            pltpu.async_copy(ytab.at[sidx[0]], rows[0], gsem[0])

            def outer(g, carry, ytab=ytab, load_idx=load_idx):
                for b in range(2):
                    j = 2 * g + b
                    b2 = 1 - b

                    @pl.when((j >= 1) & (j <= NCHK - 2))
                    def _dr(b2=b2):
                        pltpu.make_async_copy(rows[b2], acc.at[didx[b2]],
                                              ssem[b2]).wait()

                    @pl.when(j <= NCHK - 2)
                    def _pre(b2=b2, j=j):
                        load_idx(b2, j + 1)
                    pltpu.make_async_copy(ytab.at[sidx[b]], rows[b],
                                          gsem[b]).wait()

                    @pl.when(j <= NCHK - 2)
                    def _nxt(b2=b2):
                        pltpu.async_copy(ytab.at[sidx[b2]], rows[b2],
                                         gsem[b2])
                    pltpu.async_copy(rows[b], acc.at[didx[b]], ssem[b],
                                     add=True)
                return carry

            lax.fori_loop(0, NCHK // 2, outer, 0)
            pltpu.make_async_copy(rows[0], acc.at[didx[0]], ssem[0]).wait()
            pltpu.make_async_copy(rows[1], acc.at[didx[1]], ssem[1]).wait()
            if TAILE:
                off = pl.multiple_of(base + NCHK * ECH, 16)
                pltpu.sync_copy(es.at[pl.ds(off, TAILE)], sidxt)
                pltpu.sync_copy(ed.at[pl.ds(off, TAILE)], didxt)

                def adjt(r, carry2):
                    sl = pl.ds(r * 16, 16)
                    sidxt[sl] = sidxt[sl] + coff
                    return carry2

                lax.fori_loop(0, TAILE // 16, adjt, 0)
                pltpu.sync_copy(ytab.at[sidxt], rows[0].at[pl.ds(0, TAILE)])
                pltpu.sync_copy(rows[0].at[pl.ds(0, TAILE)],
                                acc.at[didxt], add=True)

            plsc.subcore_barrier()
            pltpu.sync_copy(
                acc.at[pl.ds(s * RW, RW)],
                s_out.at[pl.ds(c * NPAD + s * RW, RW)],
            )
            plsc.subcore_barrier()

    return k


def _segsum3(*args):
    return _make_segsum()(*args)


CECH = 352
CPERW = E // 32
CNCHK = CPERW // CECH
CTAIL = CPERW - CNCHK * CECH


@functools.lru_cache(maxsize=None)
def _make_count():
    out_type = [jax.ShapeDtypeStruct((2 * NPAD, 128), jnp.float32)] * 3
    scratch = [
        pltpu.VMEM((CECH,), jnp.int32),
        pltpu.VMEM((CTAIL,), jnp.int32),
        pltpu.VMEM((CECH, 128), jnp.float32),
        pltpu.VMEM_SHARED((NPAD, 128), jnp.float32),
    ]

    @functools.partial(
        pl.kernel, out_type=out_type, mesh=_mesh(), scratch_types=scratch
    )
    def k(ed0, ed1, ed2, zrow, onew, o0, o1, o2, dstb, dstbt, onesv, acc):
        c = lax.axis_index("c")
        s = lax.axis_index("s")
        pltpu.sync_copy(onew, onesv)
        eds = [ed0, ed1, ed2]
        outs = [o0, o1, o2]
        base = c * (E // 2) + s * CPERW
        for mt in range(3):
            pltpu.sync_copy(zrow, acc.at[pl.ds(s * RW, RW)])
            plsc.subcore_barrier()

            def step(j, carry, ed=eds[mt]):
                off = pl.multiple_of(base + j * CECH, 16)
                pltpu.sync_copy(ed.at[pl.ds(off, CECH)], dstb)
                pltpu.sync_copy(onesv, acc.at[dstb], add=True)
                return carry
            lax.fori_loop(0, CNCHK, step, 0)
            if CTAIL:
                off = pl.multiple_of(base + CNCHK * CECH, 16)
                pltpu.sync_copy(eds[mt].at[pl.ds(off, CTAIL)], dstbt)
                pltpu.sync_copy(onesv.at[pl.ds(0, CTAIL)],
                                acc.at[dstbt], add=True)
            plsc.subcore_barrier()
            pltpu.sync_copy(acc.at[pl.ds(s * RW, RW)],
                            outs[mt].at[pl.ds(c * NPAD + s * RW, RW)])
            plsc.subcore_barrier()

    return k


def _combine_w_body(a_ref, b_ref, bv_ref, w_ref, bo_ref):
    w_ref[0] = jnp.dot(a_ref[0], b_ref[0], preferred_element_type=jnp.float32)
    bo_ref[0] = jnp.dot(bv_ref[0], b_ref[0], preferred_element_type=jnp.float32)


def _combine_weights(A, B, bv):
    W, bo = pl.pallas_call(
        _combine_w_body,
        grid=(12,),
        in_specs=[
            pl.BlockSpec((1, D, D), lambda j: (j, 0, 0)),
            pl.BlockSpec((1, D, D), lambda j: (j, 0, 0)),
            pl.BlockSpec((1, 1, D), lambda j: (j, 0, 0)),
        ],
        out_specs=[
            pl.BlockSpec((1, D, D), lambda j: (j, 0, 0)),
            pl.BlockSpec((1, 1, D), lambda j: (j, 0, 0)),
        ],
        out_shape=[
            jax.ShapeDtypeStruct((12, D, D), jnp.float32),
            jax.ShapeDtypeStruct((12, 1, D), jnp.float32),
        ],
    )(A, B, bv.reshape(12, 1, D))
    return W, bo.reshape(12, D)


def _make_project(nz, ny):
    kk = nz + ny

    def body(x_ref, w_ref, *outs):
        acc = jnp.dot(x_ref[...], w_ref[...], preferred_element_type=jnp.float32)
        for t in range(nz):
            outs[t][...] = acc[:, t * D:(t + 1) * D]
        for t in range(ny):
            c0 = (nz + t) * D
            outs[nz + t][0] = acc[:, c0:c0 + 128]
            outs[nz + t][1] = acc[:, c0 + 128:c0 + 256]

    return pl.pallas_call(
        body,
        grid=(NB,),
        in_specs=[
            pl.BlockSpec((BN, D), lambda i: (i, 0)),
            pl.BlockSpec((D, kk * D), lambda i: (0, 0)),
        ],
        out_specs=[pl.BlockSpec((BN, D), lambda i: (i, 0))] * nz
        + [pl.BlockSpec((2, BN, 128), lambda i: (0, i, 0))] * ny,
        out_shape=[jax.ShapeDtypeStruct((N, D), jnp.float32)] * nz
        + [jax.ShapeDtypeStruct((2, N, 128), jnp.float32)] * ny,
    )


_project4 = _make_project(2, 2)
_project2 = _make_project(1, 1)


def _make_pass1(nmt):
    scale = 1.0 / nmt

    def body(*refs):
        i = pl.program_id(0)
        h_ref = refs[6 * nmt]
        st_ref = refs[6 * nmt + 1]
        scr = refs[6 * nmt + 2]
        acc = None
        for m in range(nmt):
            z, slo, shi, cl, ch, bias = refs[6 * m:6 * m + 6]
            cm = jnp.maximum(cl[...][:, 0:1] + ch[...][:, 0:1], 1.0)
            sm = jnp.concatenate([slo[...], shi[...]], axis=1)
            term = z[...] + sm / cm + bias[...]
            acc = term if acc is None else acc + term
        if nmt > 1:
            acc = acc * scale
        h_ref[...] = acc

        @pl.when(i == 0)
        def _init():
            scr[...] = jnp.zeros_like(scr)

        scr[0:1, :] += jnp.sum(acc, axis=0, keepdims=True)
        scr[1:2, :] += jnp.sum(acc * acc, axis=0, keepdims=True)

        @pl.when(i == NB - 1)
        def _fin():
            st_ref[...] = scr[...]

    in_specs = []
    for _ in range(nmt):
        in_specs += [
            pl.BlockSpec((BN, D), lambda i: (i, 0)),
            pl.BlockSpec((BN, 128), lambda i: (i, 0)),
            pl.BlockSpec((BN, 128), lambda i: (i, 0)),
            pl.BlockSpec((BN, 128), lambda i: (i, 0)),
            pl.BlockSpec((BN, 128), lambda i: (i, 0)),
            pl.BlockSpec((1, D), lambda i: (0, 0)),
        ]
    return pl.pallas_call(
        body,
        grid=(NB,),
        in_specs=in_specs,
        out_specs=[
            pl.BlockSpec((BN, D), lambda i: (i, 0)),
            pl.BlockSpec((8, D), lambda i: (0, 0)),
        ],
        out_shape=[
            jax.ShapeDtypeStruct((N, D), jnp.float32),
            jax.ShapeDtypeStruct((8, D), jnp.float32),
        ],
        scratch_shapes=[pltpu.VMEM((8, D), jnp.float32)],
    )


_pass1_1 = _make_pass1(1)
_pass1_2 = _make_pass1(2)


def _bn_act(h, st_ref, g_ref, b_ref):
    st = st_ref[...]
    mu = st[0:1, :] * (1.0 / N)
    var = st[1:2, :] * (1.0 / N) - mu * mu
    inv = lax.rsqrt(var + 1.0)
    xn = (h - mu) * (inv * g_ref[...]) + b_ref[...]
    return jnp.where(xn >= 0, xn, 0.01 * xn)


def _p2mid_body(h_ref, st_ref, g_ref, b_ref, o_ref):
    o_ref[...] = _bn_act(h_ref[...], st_ref, g_ref, b_ref)


_pass2_mid = pl.pallas_call(
    _p2mid_body,
    grid=(NB,),
    in_specs=[
        pl.BlockSpec((BN, D), lambda i: (i, 0)),
        pl.BlockSpec((8, D), lambda i: (0, 0)),
        pl.BlockSpec((1, D), lambda i: (0, 0)),
        pl.BlockSpec((1, D), lambda i: (0, 0)),
    ],
    out_specs=pl.BlockSpec((BN, D), lambda i: (i, 0)),
    out_shape=jax.ShapeDtypeStruct((N, D), jnp.float32),
)


def _p2fin_body(h_ref, st_ref, g_ref, b_ref, wfc_ref, bfc_ref, o_ref):
    act = _bn_act(h_ref[...], st_ref, g_ref, b_ref)
    o_ref[...] = (
        jnp.sum(act * wfc_ref[...], axis=1, keepdims=True) + bfc_ref[...]
    )


_pass2_fin = pl.pallas_call(
    _p2fin_body,
    grid=(NB,),
    in_specs=[
        pl.BlockSpec((BN, D), lambda i: (i, 0)),
        pl.BlockSpec((8, D), lambda i: (0, 0)),
        pl.BlockSpec((1, D), lambda i: (0, 0)),
        pl.BlockSpec((1, D), lambda i: (0, 0)),
        pl.BlockSpec((1, D), lambda i: (0, 0)),
        pl.BlockSpec((1, 1), lambda i: (0, 0)),
    ],
    out_specs=pl.BlockSpec((BN, 1), lambda i: (i, 0)),
    out_shape=jax.ShapeDtypeStruct((N, 1), jnp.float32),
)


def kernel(x_n0, x_n1, edge_index_mt0, edge_index_mt1, edge_index_mt2, Wdst_0_0, bdst_0_0, Wsrc_0_0, bsrc_0_0, Wupd_0_0, bupd_0_0, Wdst_0_1, bdst_0_1, Wsrc_0_1, bsrc_0_1, Wupd_0_1, bupd_0_1, Wdst_0_2, bdst_0_2, Wsrc_0_2, bsrc_0_2, Wupd_0_2, bupd_0_2, gamma_0_n0, beta_0_n0, gamma_0_n1, beta_0_n1, Wdst_1_0, bdst_1_0, Wsrc_1_0, bsrc_1_0, Wupd_1_0, bupd_1_0, Wdst_1_1, bdst_1_1, Wsrc_1_1, bsrc_1_1, Wupd_1_1, bupd_1_1, Wdst_1_2, bdst_1_2, Wsrc_1_2, bsrc_1_2, Wupd_1_2, bupd_1_2, gamma_1_n0, beta_1_n0, gamma_1_n1, beta_1_n1, Wfc_n0, bfc_n0, Wfc_n1, bfc_n1):
    Wdst = [[Wdst_0_0, Wdst_0_1, Wdst_0_2], [Wdst_1_0, Wdst_1_1, Wdst_1_2]]
    Wsrc = [[Wsrc_0_0, Wsrc_0_1, Wsrc_0_2], [Wsrc_1_0, Wsrc_1_1, Wsrc_1_2]]
    Wupd = [[Wupd_0_0, Wupd_0_1, Wupd_0_2], [Wupd_1_0, Wupd_1_1, Wupd_1_2]]
    bdst = [[bdst_0_0, bdst_0_1, bdst_0_2], [bdst_1_0, bdst_1_1, bdst_1_2]]
    bsrc = [[bsrc_0_0, bsrc_0_1, bsrc_0_2], [bsrc_1_0, bsrc_1_1, bsrc_1_2]]
    bupd = [[bupd_0_0, bupd_0_1, bupd_0_2], [bupd_1_0, bupd_1_1, bupd_1_2]]
    gamma = [[gamma_0_n0, gamma_0_n1], [gamma_1_n0, gamma_1_n1]]
    beta = [[beta_0_n0, beta_0_n1], [beta_1_n0, beta_1_n1]]
    edges = [edge_index_mt0, edge_index_mt1, edge_index_mt2]

    A = jnp.stack([w for i in range(2) for m in range(3)
                   for w in (Wdst[i][m], Wsrc[i][m])])
    B = jnp.stack([h for i in range(2) for m in range(3)
                   for h in (Wupd[i][m][:D], Wupd[i][m][D:])])
    bv = jnp.stack([b for i in range(2) for m in range(3)
                    for b in (bdst[i][m], bsrc[i][m])])
    Wc, bo = _combine_weights(A, B, bv)

    def W1(i, m):
        return Wc[2 * (3 * i + m)]

    def W2(i, m):
        return Wc[2 * (3 * i + m) + 1]

    def bias(i, m):
        j = 2 * (3 * i + m)
        return (bo[j] + bo[j + 1] + bupd[i][m]).reshape(1, D)

    e_src = [e[0] for e in edges]
    e_dst = [e[1] for e in edges]
    zrow = jnp.zeros((RW, 128), jnp.float32)
    onew = jnp.ones((CECH, 128), jnp.float32)

    xs = {"n0": x_n0, "n1": x_n1}
    cnts = _make_count()(e_dst[0], e_dst[1], e_dst[2], zrow, onew)
    cntl = [cc[:N] for cc in cnts]
    cnth = [cc[NPAD:NPAD + N] for cc in cnts]
    for i in range(2):
        wc0 = jnp.concatenate(
            [W1(i, 1), W1(i, 2), W2(i, 0), W2(i, 2)], axis=1)
        wc1 = jnp.concatenate([W1(i, 0), W2(i, 1)], axis=1)
        z1, z2, y0, y2 = _project4(xs["n0"], wc0)
        z0, y1 = _project2(xs["n1"], wc1)
        zz = [z0, z1, z2]
        yy = [y0, y1, y2]
        yt = [y.reshape(2 * N, 128) for y in yy]
        Ss = _segsum3(yt[0], yt[1], yt[2],
                      e_src[0], e_src[1], e_src[2],
                      e_dst[0], e_dst[1], e_dst[2], zrow)
        Slo = [S[:N] for S in Ss]
        Shi = [S[NPAD:NPAD + N] for S in Ss]

        h1, st1 = _pass1_1(zz[0], Slo[0], Shi[0], cntl[0], cnth[0],
                           bias(i, 0))
        h0, st0 = _pass1_2(zz[1], Slo[1], Shi[1], cntl[1], cnth[1],
                           bias(i, 1),
                           zz[2], Slo[2], Shi[2], cntl[2], cnth[2],
                           bias(i, 2))
        g0 = gamma[i][0].reshape(1, D)
        b0 = beta[i][0].reshape(1, D)
        g1 = gamma[i][1].reshape(1, D)
        b1 = beta[i][1].reshape(1, D)
        if i == 0:
            xs = {"n0": _pass2_mid(h0, st0, g0, b0),
                  "n1": _pass2_mid(h1, st1, g1, b1)}
        else:
            out0 = _pass2_fin(h0, st0, g0, b0,
                              Wfc_n0.reshape(1, D), bfc_n0.reshape(1, 1))
            out1 = _pass2_fin(h1, st1, g1, b1,
                              Wfc_n1.reshape(1, D), bfc_n1.reshape(1, 1))
    return jnp.concatenate([out0, out1], axis=0)


# speedup vs baseline: 1.2200x; 1.2200x over previous
import functools

import jax
import jax.numpy as jnp
from jax import lax
from jax.experimental import pallas as pl
from jax.experimental.pallas import tpu as pltpu
from jax.experimental.pallas import tpu_sc as plsc

N = 10000
D = 256
E = 160000
BN = 400
NB = N // BN
NPAD = 10112
RW = NPAD // 16
PERW = E // 16

def _mesh():
    return plsc.VectorSubcoreMesh(
        core_axis_name="c", subcore_axis_name="s", num_cores=2, num_subcores=16
    )


ECH = 176
NCHK = PERW // ECH
TAILE = PERW - NCHK * ECH


@functools.lru_cache(maxsize=None)
def _make_segsum():
    out_type = [jax.ShapeDtypeStruct((2 * NPAD, 128), jnp.float32)]
    scratch = (
        [pltpu.VMEM((ECH,), jnp.int32) for _ in range(2)]
        + [pltpu.VMEM((ECH,), jnp.int32) for _ in range(2)]
        + [pltpu.VMEM((ECH, 128), jnp.float32) for _ in range(2)]
        + [pltpu.VMEM((TAILE,), jnp.int32) for _ in range(2)]
        + [pltpu.SemaphoreType.DMA for _ in range(4)]
        + [pltpu.VMEM_SHARED((NPAD, 128), jnp.float32)]
    )

    @functools.partial(
        pl.kernel, out_type=out_type, mesh=_mesh(), scratch_types=scratch
    )
    def k(ytab, es, ed, zrow, s_out, *rest):
        sidx = rest[0:2]
        didx = rest[2:4]
        rows = rest[4:6]
        sidxt, didxt = rest[6:8]
        gsem = rest[8:10]
        ssem = rest[10:12]
        acc = rest[12]
        c = lax.axis_index("c")
        s = lax.axis_index("s")
        pltpu.sync_copy(zrow, acc.at[pl.ds(s * RW, RW)])
        plsc.subcore_barrier()
        base = s * PERW
        coff = jnp.zeros((16,), jnp.int32) + c * N

        def load_idx(bb, j):
            off = pl.multiple_of(base + j * ECH, 16)
            pltpu.sync_copy(es.at[pl.ds(off, ECH)], sidx[bb])
            pltpu.sync_copy(ed.at[pl.ds(off, ECH)], didx[bb])

            def adj(r, carry2):
                sl = pl.ds(r * 16, 16)
                sidx[bb][sl] = sidx[bb][sl] + coff
                return carry2

            lax.fori_loop(0, ECH // 16, adj, 0)

        load_idx(0, 0)
        pltpu.async_copy(ytab.at[sidx[0]], rows[0], gsem[0])

        def outer(g, carry):
            for b in range(2):
                j = 2 * g + b
                b2 = 1 - b

                @pl.when((j >= 1) & (j <= NCHK - 2))
                def _dr(b2=b2):
                    pltpu.make_async_copy(rows[b2], acc.at[didx[b2]],
                                          ssem[b2]).wait()

                @pl.when(j <= NCHK - 2)
                def _pre(b2=b2, j=j):
                    load_idx(b2, j + 1)
                pltpu.make_async_copy(ytab.at[sidx[b]], rows[b],
                                      gsem[b]).wait()

                @pl.when(j <= NCHK - 2)
                def _nxt(b2=b2):
                    pltpu.async_copy(ytab.at[sidx[b2]], rows[b2], gsem[b2])
                pltpu.async_copy(rows[b], acc.at[didx[b]], ssem[b],
                                 add=True)
            return carry

        lax.fori_loop(0, NCHK // 2, outer, 0)
        pltpu.make_async_copy(rows[0], acc.at[didx[0]], ssem[0]).wait()
        pltpu.make_async_copy(rows[1], acc.at[didx[1]], ssem[1]).wait()
        if TAILE:
            off = pl.multiple_of(base + NCHK * ECH, 16)
            pltpu.sync_copy(es.at[pl.ds(off, TAILE)], sidxt)
            pltpu.sync_copy(ed.at[pl.ds(off, TAILE)], didxt)

            def adjt(r, carry2):
                sl = pl.ds(r * 16, 16)
                sidxt[sl] = sidxt[sl] + coff
                return carry2

            lax.fori_loop(0, TAILE // 16, adjt, 0)
            pltpu.sync_copy(ytab.at[sidxt], rows[0].at[pl.ds(0, TAILE)])
            pltpu.sync_copy(rows[0].at[pl.ds(0, TAILE)],
                            acc.at[didxt], add=True)

        plsc.subcore_barrier()
        pltpu.sync_copy(
            acc.at[pl.ds(s * RW, RW)],
            s_out.at[pl.ds(c * NPAD + s * RW, RW)],
        )

    return k


def _segsum(*args):
    res = _make_segsum()(*args)
    return res[0] if isinstance(res, (list, tuple)) else res


CECH = 352
CPERW = E // 32
CNCHK = CPERW // CECH
CTAIL = CPERW - CNCHK * CECH


@functools.lru_cache(maxsize=None)
def _make_count():
    out_type = [jax.ShapeDtypeStruct((2 * NPAD, 128), jnp.float32)] * 3
    scratch = [
        pltpu.VMEM((CECH,), jnp.int32),
        pltpu.VMEM((CTAIL,), jnp.int32),
        pltpu.VMEM((CECH, 128), jnp.float32),
        pltpu.VMEM_SHARED((NPAD, 128), jnp.float32),
    ]

    @functools.partial(
        pl.kernel, out_type=out_type, mesh=_mesh(), scratch_types=scratch
    )
    def k(ed0, ed1, ed2, zrow, onew, o0, o1, o2, dstb, dstbt, onesv, acc):
        c = lax.axis_index("c")
        s = lax.axis_index("s")
        pltpu.sync_copy(onew, onesv)
        eds = [ed0, ed1, ed2]
        outs = [o0, o1, o2]
        base = c * (E // 2) + s * CPERW
        for mt in range(3):
            pltpu.sync_copy(zrow, acc.at[pl.ds(s * RW, RW)])
            plsc.subcore_barrier()

            def step(j, carry, ed=eds[mt]):
                off = pl.multiple_of(base + j * CECH, 16)
                pltpu.sync_copy(ed.at[pl.ds(off, CECH)], dstb)
                pltpu.sync_copy(onesv, acc.at[dstb], add=True)
                return carry
            lax.fori_loop(0, CNCHK, step, 0)
            if CTAIL:
                off = pl.multiple_of(base + CNCHK * CECH, 16)
                pltpu.sync_copy(eds[mt].at[pl.ds(off, CTAIL)], dstbt)
                pltpu.sync_copy(onesv.at[pl.ds(0, CTAIL)],
                                acc.at[dstbt], add=True)
            plsc.subcore_barrier()
            pltpu.sync_copy(acc.at[pl.ds(s * RW, RW)],
                            outs[mt].at[pl.ds(c * NPAD + s * RW, RW)])
            plsc.subcore_barrier()

    return k


def _combine_w_body(a_ref, b_ref, bv_ref, w_ref, bo_ref):
    w_ref[0] = jnp.dot(a_ref[0], b_ref[0], preferred_element_type=jnp.float32)
    bo_ref[0] = jnp.dot(bv_ref[0], b_ref[0], preferred_element_type=jnp.float32)


def _combine_weights(A, B, bv):
    W, bo = pl.pallas_call(
        _combine_w_body,
        grid=(12,),
        in_specs=[
            pl.BlockSpec((1, D, D), lambda j: (j, 0, 0)),
            pl.BlockSpec((1, D, D), lambda j: (j, 0, 0)),
            pl.BlockSpec((1, 1, D), lambda j: (j, 0, 0)),
        ],
        out_specs=[
            pl.BlockSpec((1, D, D), lambda j: (j, 0, 0)),
            pl.BlockSpec((1, 1, D), lambda j: (j, 0, 0)),
        ],
        out_shape=[
            jax.ShapeDtypeStruct((12, D, D), jnp.float32),
            jax.ShapeDtypeStruct((12, 1, D), jnp.float32),
        ],
    )(A, B, bv.reshape(12, 1, D))
    return W, bo.reshape(12, D)


def _make_project(nz, ny):
    kk = nz + ny

    def body(x_ref, w_ref, *outs):
        acc = jnp.dot(x_ref[...], w_ref[...], preferred_element_type=jnp.float32)
        for t in range(nz):
            outs[t][...] = acc[:, t * D:(t + 1) * D]
        for t in range(ny):
            c0 = (nz + t) * D
            outs[nz + t][0] = acc[:, c0:c0 + 128]
            outs[nz + t][1] = acc[:, c0 + 128:c0 + 256]

    return pl.pallas_call(
        body,
        grid=(NB,),
        in_specs=[
            pl.BlockSpec((BN, D), lambda i: (i, 0)),
            pl.BlockSpec((D, kk * D), lambda i: (0, 0)),
        ],
        out_specs=[pl.BlockSpec((BN, D), lambda i: (i, 0))] * nz
        + [pl.BlockSpec((2, BN, 128), lambda i: (0, i, 0))] * ny,
        out_shape=[jax.ShapeDtypeStruct((N, D), jnp.float32)] * nz
        + [jax.ShapeDtypeStruct((2, N, 128), jnp.float32)] * ny,
    )


_project4 = _make_project(2, 2)
_project2 = _make_project(1, 1)


def _make_pass1(nmt):
    scale = 1.0 / nmt

    def body(*refs):
        i = pl.program_id(0)
        h_ref = refs[6 * nmt]
        st_ref = refs[6 * nmt + 1]
        scr = refs[6 * nmt + 2]
        acc = None
        for m in range(nmt):
            z, slo, shi, cl, ch, bias = refs[6 * m:6 * m + 6]
            cm = jnp.maximum(cl[...][:, 0:1] + ch[...][:, 0:1], 1.0)
            sm = jnp.concatenate([slo[...], shi[...]], axis=1)
            term = z[...] + sm / cm + bias[...]
            acc = term if acc is None else acc + term
        if nmt > 1:
            acc = acc * scale
        h_ref[...] = acc

        @pl.when(i == 0)
        def _init():
            scr[...] = jnp.zeros_like(scr)

        scr[0:1, :] += jnp.sum(acc, axis=0, keepdims=True)
        scr[1:2, :] += jnp.sum(acc * acc, axis=0, keepdims=True)

        @pl.when(i == NB - 1)
        def _fin():
            st_ref[...] = scr[...]

    in_specs = []
    for _ in range(nmt):
        in_specs += [
            pl.BlockSpec((BN, D), lambda i: (i, 0)),
            pl.BlockSpec((BN, 128), lambda i: (i, 0)),
            pl.BlockSpec((BN, 128), lambda i: (i, 0)),
            pl.BlockSpec((BN, 128), lambda i: (i, 0)),
            pl.BlockSpec((BN, 128), lambda i: (i, 0)),
            pl.BlockSpec((1, D), lambda i: (0, 0)),
        ]
    return pl.pallas_call(
        body,
        grid=(NB,),
        in_specs=in_specs,
        out_specs=[
            pl.BlockSpec((BN, D), lambda i: (i, 0)),
            pl.BlockSpec((8, D), lambda i: (0, 0)),
        ],
        out_shape=[
            jax.ShapeDtypeStruct((N, D), jnp.float32),
            jax.ShapeDtypeStruct((8, D), jnp.float32),
        ],
        scratch_shapes=[pltpu.VMEM((8, D), jnp.float32)],
    )


_pass1_1 = _make_pass1(1)
_pass1_2 = _make_pass1(2)


def _bn_act(h, st_ref, g_ref, b_ref):
    st = st_ref[...]
    mu = st[0:1, :] * (1.0 / N)
    var = st[1:2, :] * (1.0 / N) - mu * mu
    inv = lax.rsqrt(var + 1.0)
    xn = (h - mu) * (inv * g_ref[...]) + b_ref[...]
    return jnp.where(xn >= 0, xn, 0.01 * xn)


def _p2mid_body(h_ref, st_ref, g_ref, b_ref, o_ref):
    o_ref[...] = _bn_act(h_ref[...], st_ref, g_ref, b_ref)


_pass2_mid = pl.pallas_call(
    _p2mid_body,
    grid=(NB,),
    in_specs=[
        pl.BlockSpec((BN, D), lambda i: (i, 0)),
        pl.BlockSpec((8, D), lambda i: (0, 0)),
        pl.BlockSpec((1, D), lambda i: (0, 0)),
        pl.BlockSpec((1, D), lambda i: (0, 0)),
    ],
    out_specs=pl.BlockSpec((BN, D), lambda i: (i, 0)),
    out_shape=jax.ShapeDtypeStruct((N, D), jnp.float32),
)


def _p2fin_body(h_ref, st_ref, g_ref, b_ref, wfc_ref, bfc_ref, o_ref):
    act = _bn_act(h_ref[...], st_ref, g_ref, b_ref)
    o_ref[...] = (
        jnp.sum(act * wfc_ref[...], axis=1, keepdims=True) + bfc_ref[...]
    )


_pass2_fin = pl.pallas_call(
    _p2fin_body,
    grid=(NB,),
    in_specs=[
        pl.BlockSpec((BN, D), lambda i: (i, 0)),
        pl.BlockSpec((8, D), lambda i: (0, 0)),
        pl.BlockSpec((1, D), lambda i: (0, 0)),
        pl.BlockSpec((1, D), lambda i: (0, 0)),
        pl.BlockSpec((1, D), lambda i: (0, 0)),
        pl.BlockSpec((1, 1), lambda i: (0, 0)),
    ],
    out_specs=pl.BlockSpec((BN, 1), lambda i: (i, 0)),
    out_shape=jax.ShapeDtypeStruct((N, 1), jnp.float32),
)


def kernel(x_n0, x_n1, edge_index_mt0, edge_index_mt1, edge_index_mt2, Wdst_0_0, bdst_0_0, Wsrc_0_0, bsrc_0_0, Wupd_0_0, bupd_0_0, Wdst_0_1, bdst_0_1, Wsrc_0_1, bsrc_0_1, Wupd_0_1, bupd_0_1, Wdst_0_2, bdst_0_2, Wsrc_0_2, bsrc_0_2, Wupd_0_2, bupd_0_2, gamma_0_n0, beta_0_n0, gamma_0_n1, beta_0_n1, Wdst_1_0, bdst_1_0, Wsrc_1_0, bsrc_1_0, Wupd_1_0, bupd_1_0, Wdst_1_1, bdst_1_1, Wsrc_1_1, bsrc_1_1, Wupd_1_1, bupd_1_1, Wdst_1_2, bdst_1_2, Wsrc_1_2, bsrc_1_2, Wupd_1_2, bupd_1_2, gamma_1_n0, beta_1_n0, gamma_1_n1, beta_1_n1, Wfc_n0, bfc_n0, Wfc_n1, bfc_n1):
    Wdst = [[Wdst_0_0, Wdst_0_1, Wdst_0_2], [Wdst_1_0, Wdst_1_1, Wdst_1_2]]
    Wsrc = [[Wsrc_0_0, Wsrc_0_1, Wsrc_0_2], [Wsrc_1_0, Wsrc_1_1, Wsrc_1_2]]
    Wupd = [[Wupd_0_0, Wupd_0_1, Wupd_0_2], [Wupd_1_0, Wupd_1_1, Wupd_1_2]]
    bdst = [[bdst_0_0, bdst_0_1, bdst_0_2], [bdst_1_0, bdst_1_1, bdst_1_2]]
    bsrc = [[bsrc_0_0, bsrc_0_1, bsrc_0_2], [bsrc_1_0, bsrc_1_1, bsrc_1_2]]
    bupd = [[bupd_0_0, bupd_0_1, bupd_0_2], [bupd_1_0, bupd_1_1, bupd_1_2]]
    gamma = [[gamma_0_n0, gamma_0_n1], [gamma_1_n0, gamma_1_n1]]
    beta = [[beta_0_n0, beta_0_n1], [beta_1_n0, beta_1_n1]]
    edges = [edge_index_mt0, edge_index_mt1, edge_index_mt2]

    A = jnp.stack([w for i in range(2) for m in range(3)
                   for w in (Wdst[i][m], Wsrc[i][m])])
    B = jnp.stack([h for i in range(2) for m in range(3)
                   for h in (Wupd[i][m][:D], Wupd[i][m][D:])])
    bv = jnp.stack([b for i in range(2) for m in range(3)
                    for b in (bdst[i][m], bsrc[i][m])])
    Wc, bo = _combine_weights(A, B, bv)

    def W1(i, m):
        return Wc[2 * (3 * i + m)]

    def W2(i, m):
        return Wc[2 * (3 * i + m) + 1]

    def bias(i, m):
        j = 2 * (3 * i + m)
        return (bo[j] + bo[j + 1] + bupd[i][m]).reshape(1, D)

    e_src = [e[0] for e in edges]
    e_dst = [e[1] for e in edges]
    zrow = jnp.zeros((RW, 128), jnp.float32)
    onew = jnp.ones((CECH, 128), jnp.float32)

    xs = {"n0": x_n0, "n1": x_n1}
    cnts = _make_count()(e_dst[0], e_dst[1], e_dst[2], zrow, onew)
    cntl = [cc[:N] for cc in cnts]
    cnth = [cc[NPAD:NPAD + N] for cc in cnts]
    for i in range(2):
        wc0 = jnp.concatenate(
            [W1(i, 1), W1(i, 2), W2(i, 0), W2(i, 2)], axis=1)
        wc1 = jnp.concatenate([W1(i, 0), W2(i, 1)], axis=1)
        z1, z2, y0, y2 = _project4(xs["n0"], wc0)
        z0, y1 = _project2(xs["n1"], wc1)
        zz = [z0, z1, z2]
        yy = [y0, y1, y2]
        Slo = [None, None, None]
        Shi = [None, None, None]
        for m in range(3):
            ytab = yy[m].reshape(2 * N, 128)
            S = _segsum(ytab, e_src[m], e_dst[m], zrow)
            Slo[m] = S[:N]
            Shi[m] = S[NPAD:NPAD + N]

        h1, st1 = _pass1_1(zz[0], Slo[0], Shi[0], cntl[0], cnth[0],
                           bias(i, 0))
        h0, st0 = _pass1_2(zz[1], Slo[1], Shi[1], cntl[1], cnth[1],
                           bias(i, 1),
                           zz[2], Slo[2], Shi[2], cntl[2], cnth[2],
                           bias(i, 2))
        g0 = gamma[i][0].reshape(1, D)
        b0 = beta[i][0].reshape(1, D)
        g1 = gamma[i][1].reshape(1, D)
        b1 = beta[i][1].reshape(1, D)
        if i == 0:
            xs = {"n0": _pass2_mid(h0, st0, g0, b0),
                  "n1": _pass2_mid(h1, st1, g1, b1)}
        else:
            out0 = _pass2_fin(h0, st0, g0, b0,
                              Wfc_n0.reshape(1, D), bfc_n0.reshape(1, 1))
            out1 = _pass2_fin(h1, st1, g1, b1,
                              Wfc_n1.reshape(1, D), bfc_n1.reshape(1, 1))
    return jnp.concatenate([out0, out1], axis=0)
